# baseline (speedup 1.0000x reference)
import functools

import jax
import jax.numpy as jnp
from jax import lax
from jax.experimental import pallas as pl
from jax.experimental.pallas import tpu as pltpu
from jax.experimental.pallas import tpu_sc as plsc

_C = 6625
_D = 512
_B = 64
_S = 80
_N = _B * _S
_SPLITS = ((0, 3840), (3840, 1280))
_NEG = -3.4e38


_R1 = 640


def _prep_body(p_ref, labels_ref, ll_ref, labs_ref, w_ref):
    x = p_ref[...]
    m = jnp.max(x, axis=1, keepdims=True)
    ci = lax.broadcasted_iota(jnp.int32, (_R1, _C), 1)
    raw = jnp.min(jnp.where(x == m, ci, _C), axis=1, keepdims=True)

    ri = lax.broadcasted_iota(jnp.int32, (_R1, 1), 0)
    prev = jnp.concatenate(
        [jnp.full((1, 1), -1, jnp.int32), raw[:-1]], axis=0)
    first = (ri % _S) == 0
    char_rep = jnp.logical_and(prev == raw, jnp.logical_not(first))
    mk = jnp.logical_and(raw > 0, jnp.logical_not(char_rep)
                         ).astype(jnp.float32)

    rr = lax.broadcasted_iota(jnp.int32, (_R1, _R1), 0)
    tt = lax.broadcasted_iota(jnp.int32, (_R1, _R1), 1)
    sameseg = (rr // _S) == (tt // _S)
    seg_f = sameseg.astype(jnp.float32)
    ltri = jnp.where(tt <= rr, seg_f, 0.0)

    cs = jnp.dot(ltri, mk, preferred_element_type=jnp.float32)
    cnt = jnp.dot(seg_f, mk, preferred_element_type=jnp.float32)

    nsamp = _R1 // _S
    ra = lax.broadcasted_iota(jnp.int32, (_R1, nsamp), 0)
    ca = lax.broadcasted_iota(jnp.int32, (_R1, nsamp), 1)
    amat = ((ra // _S) == ca).astype(jnp.float32)
    ll_col = jnp.dot(amat, ll_ref[:, :1].astype(jnp.float32),
                     preferred_element_type=jnp.float32)
    lab_rows = jnp.dot(amat, labels_ref[...].astype(jnp.float32),
                       preferred_element_type=jnp.float32)
    t80 = lax.broadcasted_iota(jnp.int32, (_R1, _S), 1)
    r80 = lax.broadcasted_iota(jnp.int32, (_R1, _S), 0) % _S
    labels_col = jnp.sum(jnp.where(t80 == r80, lab_rows, 0.0),
                         axis=1, keepdims=True)

    valid = (cnt == ll_col).astype(jnp.float32)
    rank = jnp.clip(cs.astype(jnp.int32) - 1, 0, _S - 1)

    tloc = tt % _S
    onehot = jnp.where(tloc == rank, seg_f, 0.0)
    labs_f = jnp.dot(onehot, labels_col,
                     preferred_element_type=jnp.float32)

    labs_ref[...] = labs_f.astype(jnp.int32)
    w_ref[...] = jnp.broadcast_to(mk * valid, (_R1, 16))


def _prep(predicts2, labels2, ll_rep, start, rows):
    base = start // _R1
    grid = rows // _R1
    return pl.pallas_call(
        _prep_body,
        grid=(grid,),
        in_specs=[
            pl.BlockSpec((_R1, _C), lambda r: (base + r, 0)),
            pl.BlockSpec((_R1 // _S, _S), lambda r: (base + r, 0)),
            pl.BlockSpec((_R1 // _S, _S), lambda r: (base + r, 0)),
        ],
        out_specs=[
            pl.BlockSpec((_R1, 1), lambda r: (r, 0)),
            pl.BlockSpec((_R1, 16), lambda r: (r, 0)),
        ],
        out_shape=[
            jax.ShapeDtypeStruct((rows, 1), jnp.int32),
            jax.ShapeDtypeStruct((rows, 16), jnp.float32),
        ],
    )(predicts2, labels2, ll_rep)


_NC, _NS = 2, 16
_NW = _NC * _NS
_CHUNK = 40


def _sc_body(row_base, per_w, nchunk,
             centers_hbm, labs_hbm, w_hbm, emb_hbm,
             out_sq_hbm, out_w_hbm,
             idx_v, w_v, c_v0, c_v1, e_v0, e_v1, res_v,
             sem_c0, sem_c1, sem_e0, sem_e1):
    wid = lax.axis_index("s") * _NC + lax.axis_index("c")
    base = wid * per_w
    ebase = row_base + base
    pltpu.sync_copy(labs_hbm.at[pl.ds(base, per_w)], idx_v)
    pltpu.sync_copy(w_hbm.at[pl.ds(base, per_w)], w_v)

    cbuf = (c_v0, c_v1)
    ebuf = (e_v0, e_v1)
    csem = (sem_c0, sem_c1)
    esem = (sem_e0, sem_e1)

    def start(g):
        slot = g % 2
        dc = pltpu.async_copy(
            centers_hbm.at[idx_v.at[pl.ds(g * _CHUNK, _CHUNK)]],
            cbuf[slot], csem[slot])
        de = pltpu.async_copy(
            emb_hbm.at[pl.ds(ebase + g * _CHUNK, _CHUNK)],
            ebuf[slot], esem[slot])
        return dc, de

    pend = start(0)
    acc = jnp.zeros((16,), jnp.float32)
    wacc = jnp.zeros((16,), jnp.float32)
    for g in range(nchunk):
        nxt = start(g + 1) if g + 1 < nchunk else None
        pend[0].wait()
        pend[1].wait()
        c_v = cbuf[g % 2]
        e_v = ebuf[g % 2]

        def row_body(r, carry):
            acc, wacc = carry
            wspl = w_v[g * _CHUNK + r, :]
            s = jnp.zeros((16,), jnp.float32)
            for k in range(_D // 16):
                ev = e_v[r, pl.ds(k * 16, 16)]
                cv = c_v[r, pl.ds(k * 16, 16)]
                d = ev - cv
                s = s + d * d
            return acc + wspl * s, wacc + wspl

        acc, wacc = lax.fori_loop(0, _CHUNK, row_body, (acc, wacc))
        pend = nxt

    res_v[0, :] = acc
    res_v[1, :] = wacc
    pltpu.sync_copy(res_v.at[0], out_sq_hbm.at[wid])
    pltpu.sync_copy(res_v.at[1], out_w_hbm.at[wid])


def _sc_loss(centers, labs_flat, w16, emb_flat, row_base, rows):
    per_w = rows // _NW
    nchunk = per_w // _CHUNK
    mesh = plsc.VectorSubcoreMesh(
        core_axis_name="c", subcore_axis_name="s")
    run = pl.kernel(
        functools.partial(_sc_body, row_base, per_w, nchunk),
        out_type=[
            jax.ShapeDtypeStruct((_NW, 16), jnp.float32),
            jax.ShapeDtypeStruct((_NW, 16), jnp.float32),
        ],
        mesh=mesh,
        scratch_types=[
            pltpu.VMEM((per_w,), jnp.int32),
            pltpu.VMEM((per_w, 16), jnp.float32),
            pltpu.VMEM((_CHUNK, _D), jnp.float32),
            pltpu.VMEM((_CHUNK, _D), jnp.float32),
            pltpu.VMEM((_CHUNK, _D), jnp.float32),
            pltpu.VMEM((_CHUNK, _D), jnp.float32),
            pltpu.VMEM((2, 16), jnp.float32),
            pltpu.SemaphoreType.DMA,
            pltpu.SemaphoreType.DMA,
            pltpu.SemaphoreType.DMA,
            pltpu.SemaphoreType.DMA,
        ],
    )
    return run(centers, labs_flat, w16, emb_flat)


@jax.jit
def kernel(predicts, embedding, labels, label_len, centers):
    predicts2 = predicts.reshape(_N, _C)
    ll2 = jnp.broadcast_to(label_len[:, None], (_B, _S))
    emb_flat = embedding.reshape(_N, _D)

    parts = []
    for start, rows in _SPLITS:
        labs, w16 = _prep(predicts2, labels, ll2, start, rows)
        part_sq, part_w = _sc_loss(
            centers, labs.reshape(rows), w16, emb_flat, start, rows)
        parts.append((part_sq, part_w))

    cat = jnp.concatenate([p[0] for p in parts] + [p[1] for p in parts])
    s = jnp.sum(cat.reshape(2, len(_SPLITS) * _NW * 16), axis=1)
    total = s[0]
    wsum = s[1] / 16.0
    return total / (wsum * _D)


# speedup vs baseline: 1.0852x; 1.0852x over previous
import functools

import jax
import jax.numpy as jnp
from jax import lax
from jax.experimental import pallas as pl
from jax.experimental.pallas import tpu as pltpu
from jax.experimental.pallas import tpu_sc as plsc

_C = 6625
_D = 512
_B = 64
_S = 80
_N = _B * _S
_NH = 1
_NROWS = _N // _NH
_NEG = -3.4e38


_R1 = 640
_G1 = _NROWS // _R1


def _prep_body(p_ref, labels_ref, ll_ref, labs_ref, w_ref):
    x = p_ref[...]
    m = jnp.max(x, axis=1, keepdims=True)
    ci = lax.broadcasted_iota(jnp.int32, (_R1, _C), 1)
    raw = jnp.min(jnp.where(x == m, ci, _C), axis=1, keepdims=True)

    ri = lax.broadcasted_iota(jnp.int32, (_R1, 1), 0)
    prev = jnp.concatenate(
        [jnp.full((1, 1), -1, jnp.int32), raw[:-1]], axis=0)
    first = (ri % _S) == 0
    char_rep = jnp.logical_and(prev == raw, jnp.logical_not(first))
    mk = jnp.logical_and(raw > 0, jnp.logical_not(char_rep)
                         ).astype(jnp.float32)

    rr = lax.broadcasted_iota(jnp.int32, (_R1, _R1), 0)
    tt = lax.broadcasted_iota(jnp.int32, (_R1, _R1), 1)
    sameseg = (rr // _S) == (tt // _S)
    seg_f = sameseg.astype(jnp.float32)
    ltri = jnp.where(tt <= rr, seg_f, 0.0)

    cs = jnp.dot(ltri, mk, preferred_element_type=jnp.float32)
    cnt = jnp.dot(seg_f, mk, preferred_element_type=jnp.float32)

    nsamp = _R1 // _S
    ra = lax.broadcasted_iota(jnp.int32, (_R1, nsamp), 0)
    ca = lax.broadcasted_iota(jnp.int32, (_R1, nsamp), 1)
    amat = ((ra // _S) == ca).astype(jnp.float32)
    ll_col = jnp.dot(amat, ll_ref[:, :1].astype(jnp.float32),
                     preferred_element_type=jnp.float32)
    lab_rows = jnp.dot(amat, labels_ref[...].astype(jnp.float32),
                       preferred_element_type=jnp.float32)
    t80 = lax.broadcasted_iota(jnp.int32, (_R1, _S), 1)
    r80 = lax.broadcasted_iota(jnp.int32, (_R1, _S), 0) % _S
    labels_col = jnp.sum(jnp.where(t80 == r80, lab_rows, 0.0),
                         axis=1, keepdims=True)

    valid = (cnt == ll_col).astype(jnp.float32)
    rank = jnp.clip(cs.astype(jnp.int32) - 1, 0, _S - 1)

    tloc = tt % _S
    onehot = jnp.where(tloc == rank, seg_f, 0.0)
    labs_f = jnp.dot(onehot, labels_col,
                     preferred_element_type=jnp.float32)

    ident = (rr == tt).astype(jnp.float32)
    labs_row = lax.dot_general(
        labs_f, ident, (((0,), (0,)), ((), ())),
        preferred_element_type=jnp.float32)
    labs_ref[...] = labs_row.astype(jnp.int32)
    w_ref[...] = jnp.broadcast_to(mk * valid, (_R1, 16))


def _prep(predicts2, labels2, ll_rep, half):
    base = half * _G1
    return pl.pallas_call(
        _prep_body,
        grid=(_G1,),
        in_specs=[
            pl.BlockSpec((_R1, _C), lambda r: (base + r, 0)),
            pl.BlockSpec((_R1 // _S, _S), lambda r: (base + r, 0)),
            pl.BlockSpec((_R1 // _S, _S), lambda r: (base + r, 0)),
        ],
        out_specs=[
            pl.BlockSpec((1, _R1), lambda r: (0, r)),
            pl.BlockSpec((_R1, 16), lambda r: (r, 0)),
        ],
        out_shape=[
            jax.ShapeDtypeStruct((1, _NROWS), jnp.int32),
            jax.ShapeDtypeStruct((_NROWS, 16), jnp.float32),
        ],
    )(predicts2, labels2, ll_rep)


_NC, _NS = 2, 16
_NW = _NC * _NS
_PER_W = _NROWS // _NW
_CHUNK = 32
_NCHUNK = _PER_W // _CHUNK
_NBUF = 3


def _sc_body(row_base, centers_hbm, labs_hbm, w_hbm, emb_hbm,
             out_sq_hbm, out_w_hbm,
             idx_v, w_v, c_v0, c_v1, c_v2, e_v0, e_v1, e_v2, res_v,
             sem_c0, sem_c1, sem_c2, sem_e0, sem_e1, sem_e2):
    wid = lax.axis_index("s") * _NC + lax.axis_index("c")
    base = wid * _PER_W
    ebase = row_base + base
    pltpu.sync_copy(labs_hbm.at[0], idx_v)
    pltpu.sync_copy(w_hbm.at[pl.ds(base, _PER_W)], w_v)

    cbuf = (c_v0, c_v1, c_v2)
    ebuf = (e_v0, e_v1, e_v2)
    csem = (sem_c0, sem_c1, sem_c2)
    esem = (sem_e0, sem_e1, sem_e2)

    def start(g):
        slot = g % _NBUF
        dc = pltpu.async_copy(
            centers_hbm.at[idx_v.at[pl.ds(base + g * _CHUNK, _CHUNK)]],
            cbuf[slot], csem[slot])
        de = pltpu.async_copy(
            emb_hbm.at[pl.ds(ebase + g * _CHUNK, _CHUNK)],
            ebuf[slot], esem[slot])
        return dc, de

    pend = {g: start(g) for g in range(_NBUF - 1)}
    acc = jnp.zeros((16,), jnp.float32)
    wacc = jnp.zeros((16,), jnp.float32)
    for g in range(_NCHUNK):
        if g + _NBUF - 1 < _NCHUNK:
            pend[g + _NBUF - 1] = start(g + _NBUF - 1)
        pend[g][0].wait()
        pend[g][1].wait()
        c_v = cbuf[g % _NBUF]
        e_v = ebuf[g % _NBUF]

        def row_body(r, carry):
            acc, wacc = carry
            wspl = w_v[g * _CHUNK + r, :]
            s = jnp.zeros((16,), jnp.float32)
            for k in range(_D // 16):
                ev = e_v[r, pl.ds(k * 16, 16)]
                cv = c_v[r, pl.ds(k * 16, 16)]
                d = ev - cv
                s = s + d * d
            return acc + wspl * s, wacc + wspl

        acc, wacc = lax.fori_loop(0, _CHUNK, row_body, (acc, wacc))

    res_v[0, :] = acc
    res_v[1, :] = wacc
    pltpu.sync_copy(res_v.at[0], out_sq_hbm.at[wid])
    pltpu.sync_copy(res_v.at[1], out_w_hbm.at[wid])


def _sc_loss(centers, labs_row, w16, emb_flat, row_base):
    mesh = plsc.VectorSubcoreMesh(
        core_axis_name="c", subcore_axis_name="s")
    run = pl.kernel(
        functools.partial(_sc_body, row_base),
        out_type=[
            jax.ShapeDtypeStruct((_NW, 16), jnp.float32),
            jax.ShapeDtypeStruct((_NW, 16), jnp.float32),
        ],
        mesh=mesh,
        scratch_types=[
            pltpu.VMEM((_NROWS,), jnp.int32),
            pltpu.VMEM((_PER_W, 16), jnp.float32),
            pltpu.VMEM((_CHUNK, _D), jnp.float32),
            pltpu.VMEM((_CHUNK, _D), jnp.float32),
            pltpu.VMEM((_CHUNK, _D), jnp.float32),
            pltpu.VMEM((_CHUNK, _D), jnp.float32),
            pltpu.VMEM((_CHUNK, _D), jnp.float32),
            pltpu.VMEM((_CHUNK, _D), jnp.float32),
            pltpu.VMEM((2, 16), jnp.float32),
            pltpu.SemaphoreType.DMA,
            pltpu.SemaphoreType.DMA,
            pltpu.SemaphoreType.DMA,
            pltpu.SemaphoreType.DMA,
            pltpu.SemaphoreType.DMA,
            pltpu.SemaphoreType.DMA,
        ],
    )
    return run(centers, labs_row, w16, emb_flat)


@jax.jit
def kernel(predicts, embedding, labels, label_len, centers):
    predicts2 = predicts.reshape(_N, _C)
    ll2 = jnp.broadcast_to(label_len[:, None], (_B, _S))
    emb_flat = embedding.reshape(_N, _D)

    parts = []
    for h in range(_NH):
        labs, w16 = _prep(predicts2, labels, ll2, h)
        part_sq, part_w = _sc_loss(
            centers, labs, w16, emb_flat, h * _NROWS)
        parts.append((part_sq, part_w))

    cat = jnp.concatenate([p[0] for p in parts] + [p[1] for p in parts])
    s = jnp.sum(cat.reshape(2, _NH * _NW * 16 // 1), axis=1)
    total = s[0]
    wsum = s[1] / 16.0
    return total / (wsum * _D)


# speedup vs baseline: 1.1073x; 1.0204x over previous
import functools

import jax
import jax.numpy as jnp
from jax import lax
from jax.experimental import pallas as pl
from jax.experimental.pallas import tpu as pltpu
from jax.experimental.pallas import tpu_sc as plsc

_C = 6625
_D = 512
_B = 64
_S = 80
_N = _B * _S
_NH = 1
_NROWS = _N // _NH
_NEG = -3.4e38


_R1 = 640
_G1 = _NROWS // _R1


def _prep_body(p_ref, labels_ref, ll_ref, labs_ref, w_ref):
    x = p_ref[...]
    m = jnp.max(x, axis=1, keepdims=True)
    ci = lax.broadcasted_iota(jnp.int32, (_R1, _C), 1)
    raw = jnp.min(jnp.where(x == m, ci, _C), axis=1, keepdims=True)

    ri = lax.broadcasted_iota(jnp.int32, (_R1, 1), 0)
    prev = jnp.concatenate(
        [jnp.full((1, 1), -1, jnp.int32), raw[:-1]], axis=0)
    first = (ri % _S) == 0
    char_rep = jnp.logical_and(prev == raw, jnp.logical_not(first))
    mk = jnp.logical_and(raw > 0, jnp.logical_not(char_rep)
                         ).astype(jnp.float32)

    rr = lax.broadcasted_iota(jnp.int32, (_R1, _R1), 0)
    tt = lax.broadcasted_iota(jnp.int32, (_R1, _R1), 1)
    sameseg = (rr // _S) == (tt // _S)
    seg_f = sameseg.astype(jnp.float32)
    ltri = jnp.where(tt <= rr, seg_f, 0.0)

    cs = jnp.dot(ltri, mk, preferred_element_type=jnp.float32)
    cnt = jnp.dot(seg_f, mk, preferred_element_type=jnp.float32)

    nsamp = _R1 // _S
    ra = lax.broadcasted_iota(jnp.int32, (_R1, nsamp), 0)
    ca = lax.broadcasted_iota(jnp.int32, (_R1, nsamp), 1)
    amat = ((ra // _S) == ca).astype(jnp.float32)
    ll_col = jnp.dot(amat, ll_ref[:, :1].astype(jnp.float32),
                     preferred_element_type=jnp.float32)
    lab_rows = jnp.dot(amat, labels_ref[...].astype(jnp.float32),
                       preferred_element_type=jnp.float32)
    t80 = lax.broadcasted_iota(jnp.int32, (_R1, _S), 1)
    r80 = lax.broadcasted_iota(jnp.int32, (_R1, _S), 0) % _S
    labels_col = jnp.sum(jnp.where(t80 == r80, lab_rows, 0.0),
                         axis=1, keepdims=True)

    valid = (cnt == ll_col).astype(jnp.float32)
    rank = jnp.clip(cs.astype(jnp.int32) - 1, 0, _S - 1)

    tloc = tt % _S
    onehot = jnp.where(tloc == rank, seg_f, 0.0)
    labs_f = jnp.dot(onehot, labels_col,
                     preferred_element_type=jnp.float32)

    ident = (rr == tt).astype(jnp.float32)
    labs_row = lax.dot_general(
        labs_f, ident, (((0,), (0,)), ((), ())),
        preferred_element_type=jnp.float32)
    labs_ref[...] = labs_row.astype(jnp.int32)
    w_ref[...] = jnp.broadcast_to(mk * valid, (_R1, 16))


def _prep(predicts2, labels2, ll_rep, half):
    base = half * _G1
    return pl.pallas_call(
        _prep_body,
        grid=(_G1,),
        in_specs=[
            pl.BlockSpec((_R1, _C), lambda r: (base + r, 0)),
            pl.BlockSpec((_R1 // _S, _S), lambda r: (base + r, 0)),
            pl.BlockSpec((_R1 // _S, _S), lambda r: (base + r, 0)),
        ],
        out_specs=[
            pl.BlockSpec((1, _R1), lambda r: (0, r)),
            pl.BlockSpec((_R1, 16), lambda r: (r, 0)),
        ],
        out_shape=[
            jax.ShapeDtypeStruct((1, _NROWS), jnp.int32),
            jax.ShapeDtypeStruct((_NROWS, 16), jnp.float32),
        ],
    )(predicts2, labels2, ll_rep)


_NC, _NS = 2, 16
_NW = _NC * _NS
_PER_W = _NROWS // _NW
_CHUNK = 40
_NCHUNK = _PER_W // _CHUNK
_NBUF = 2


def _sc_body(row_base, centers_hbm, labs_hbm, w_hbm, emb_hbm,
             out_sq_hbm, out_w_hbm,
             idx_v, w_v, c_v0, c_v1, e_v0, e_v1, res_v,
             sem_c0, sem_c1, sem_e0, sem_e1, sem_w):
    wid = lax.axis_index("s") * _NC + lax.axis_index("c")
    base = wid * _PER_W
    ebase = row_base + base
    pltpu.sync_copy(labs_hbm.at[0], idx_v)
    w_copy = pltpu.async_copy(
        w_hbm.at[pl.ds(base, _PER_W)], w_v, sem_w)

    cbuf = (c_v0, c_v1)
    ebuf = (e_v0, e_v1)
    csem = (sem_c0, sem_c1)
    esem = (sem_e0, sem_e1)

    def start(g):
        slot = g % _NBUF
        dc = pltpu.async_copy(
            centers_hbm.at[idx_v.at[pl.ds(base + g * _CHUNK, _CHUNK)]],
            cbuf[slot], csem[slot])
        de = pltpu.async_copy(
            emb_hbm.at[pl.ds(ebase + g * _CHUNK, _CHUNK)],
            ebuf[slot], esem[slot])
        return dc, de

    pend = {g: start(g) for g in range(_NBUF - 1)}
    w_copy.wait()
    acc = jnp.zeros((16,), jnp.float32)
    wacc = jnp.zeros((16,), jnp.float32)
    for g in range(_NCHUNK):
        if g + _NBUF - 1 < _NCHUNK:
            pend[g + _NBUF - 1] = start(g + _NBUF - 1)
        pend[g][0].wait()
        pend[g][1].wait()
        c_v = cbuf[g % _NBUF]
        e_v = ebuf[g % _NBUF]

        def row_body(r, carry):
            acc, wacc = carry
            wspl = w_v[g * _CHUNK + r, :]
            s = jnp.zeros((16,), jnp.float32)
            for k in range(_D // 16):
                ev = e_v[r, pl.ds(k * 16, 16)]
                cv = c_v[r, pl.ds(k * 16, 16)]
                d = ev - cv
                s = s + d * d
            return acc + wspl * s, wacc + wspl

        acc, wacc = lax.fori_loop(0, _CHUNK, row_body, (acc, wacc))

    res_v[0, :] = acc
    res_v[1, :] = wacc
    pltpu.sync_copy(res_v.at[0], out_sq_hbm.at[wid])
    pltpu.sync_copy(res_v.at[1], out_w_hbm.at[wid])


def _sc_loss(centers, labs_row, w16, emb_flat, row_base):
    mesh = plsc.VectorSubcoreMesh(
        core_axis_name="c", subcore_axis_name="s")
    run = pl.kernel(
        functools.partial(_sc_body, row_base),
        out_type=[
            jax.ShapeDtypeStruct((_NW, 16), jnp.float32),
            jax.ShapeDtypeStruct((_NW, 16), jnp.float32),
        ],
        mesh=mesh,
        scratch_types=[
            pltpu.VMEM((_NROWS,), jnp.int32),
            pltpu.VMEM((_PER_W, 16), jnp.float32),
            pltpu.VMEM((_CHUNK, _D), jnp.float32),
            pltpu.VMEM((_CHUNK, _D), jnp.float32),
            pltpu.VMEM((_CHUNK, _D), jnp.float32),
            pltpu.VMEM((_CHUNK, _D), jnp.float32),
            pltpu.VMEM((2, 16), jnp.float32),
            pltpu.SemaphoreType.DMA,
            pltpu.SemaphoreType.DMA,
            pltpu.SemaphoreType.DMA,
            pltpu.SemaphoreType.DMA,
            pltpu.SemaphoreType.DMA,
        ],
    )
    return run(centers, labs_row, w16, emb_flat)


@jax.jit
def kernel(predicts, embedding, labels, label_len, centers):
    predicts2 = predicts.reshape(_N, _C)
    ll2 = jnp.broadcast_to(label_len[:, None], (_B, _S))
    emb_flat = embedding.reshape(_N, _D)

    parts = []
    for h in range(_NH):
        labs, w16 = _prep(predicts2, labels, ll2, h)
        part_sq, part_w = _sc_loss(
            centers, labs, w16, emb_flat, h * _NROWS)
        parts.append((part_sq, part_w))

    cat = jnp.concatenate([p[0] for p in parts] + [p[1] for p in parts])
    s = jnp.sum(cat.reshape(2, _NH * _NW * 16 // 1), axis=1)
    total = s[0]
    wsum = s[1] / 16.0
    return total / (wsum * _D)


# speedup vs baseline: 1.1205x; 1.0119x over previous
import functools

import jax
import jax.numpy as jnp
from jax import lax
from jax.experimental import pallas as pl
from jax.experimental.pallas import tpu as pltpu
from jax.experimental.pallas import tpu_sc as plsc

_C = 6625
_D = 512
_B = 64
_S = 80
_N = _B * _S
_NH = 1
_NROWS = _N // _NH
_NEG = -3.4e38


_R1 = 640
_G1 = _NROWS // _R1


def _prep_body(ll_ref, p_ref, labels_ref, labs_ref, w_ref):
    step = pl.program_id(0)
    x = p_ref[...]
    m = jnp.max(x, axis=1, keepdims=True)
    ci = lax.broadcasted_iota(jnp.int32, (_R1, _C), 1)
    raw = jnp.min(jnp.where(x == m, ci, _C), axis=1, keepdims=True)

    ri = lax.broadcasted_iota(jnp.int32, (_R1, 1), 0)
    prev = jnp.concatenate(
        [jnp.full((1, 1), -1, jnp.int32), raw[:-1]], axis=0)
    first = (ri % _S) == 0
    char_rep = jnp.logical_and(prev == raw, jnp.logical_not(first))
    mk = jnp.logical_and(raw > 0, jnp.logical_not(char_rep)
                         ).astype(jnp.float32)

    rr = lax.broadcasted_iota(jnp.int32, (_R1, _R1), 0)
    tt = lax.broadcasted_iota(jnp.int32, (_R1, _R1), 1)
    sameseg = (rr // _S) == (tt // _S)
    seg_f = sameseg.astype(jnp.float32)
    ltri = jnp.where(tt <= rr, seg_f, 0.0)

    cs = jnp.dot(ltri, mk, preferred_element_type=jnp.float32)
    cnt = jnp.dot(seg_f, mk, preferred_element_type=jnp.float32)

    nsamp = _R1 // _S
    ra = lax.broadcasted_iota(jnp.int32, (_R1, nsamp), 0)
    ca = lax.broadcasted_iota(jnp.int32, (_R1, nsamp), 1)
    amat = ((ra // _S) == ca).astype(jnp.float32)
    ll_col = jnp.concatenate(
        [jnp.full((_S, 1), 1.0, jnp.float32)
         * ll_ref[step * nsamp + i].astype(jnp.float32)
         for i in range(nsamp)], axis=0)
    lab_rows = jnp.dot(amat, labels_ref[...].astype(jnp.float32),
                       preferred_element_type=jnp.float32)
    t80 = lax.broadcasted_iota(jnp.int32, (_R1, _S), 1)
    r80 = lax.broadcasted_iota(jnp.int32, (_R1, _S), 0) % _S
    labels_col = jnp.sum(jnp.where(t80 == r80, lab_rows, 0.0),
                         axis=1, keepdims=True)

    valid = (cnt == ll_col).astype(jnp.float32)
    rank = jnp.clip(cs.astype(jnp.int32) - 1, 0, _S - 1)

    tloc = tt % _S
    onehot = jnp.where(tloc == rank, seg_f, 0.0)
    labs_f = jnp.dot(onehot, labels_col,
                     preferred_element_type=jnp.float32)

    ident = (rr == tt).astype(jnp.float32)
    labs_row = lax.dot_general(
        labs_f, ident, (((0,), (0,)), ((), ())),
        preferred_element_type=jnp.float32)
    labs_ref[...] = labs_row.astype(jnp.int32)
    w_ref[...] = jnp.broadcast_to(mk * valid, (_R1, 16))


def _prep(predicts2, labels2, ll_rep, half):
    base = half * _G1
    return pl.pallas_call(
        _prep_body,
        grid=(_G1,),
        in_specs=[
            pl.BlockSpec(memory_space=pltpu.SMEM),
            pl.BlockSpec((_R1, _C), lambda r: (base + r, 0)),
            pl.BlockSpec((_R1 // _S, _S), lambda r: (base + r, 0)),
        ],
        out_specs=[
            pl.BlockSpec((1, _R1), lambda r: (0, r)),
            pl.BlockSpec((_R1, 16), lambda r: (r, 0)),
        ],
        out_shape=[
            jax.ShapeDtypeStruct((1, _NROWS), jnp.int32),
            jax.ShapeDtypeStruct((_NROWS, 16), jnp.float32),
        ],
    )(ll_rep, predicts2, labels2)


_NC, _NS = 2, 16
_NW = _NC * _NS
_PER_W = _NROWS // _NW
_CHUNK = 40
_NCHUNK = _PER_W // _CHUNK
_NBUF = 2


def _sc_body(row_base, centers_hbm, labs_hbm, w_hbm, emb_hbm,
             out_sq_hbm, out_w_hbm,
             idx_v, w_v, c_v0, c_v1, e_v0, e_v1, res_v,
             sem_c0, sem_c1, sem_e0, sem_e1, sem_w):
    wid = lax.axis_index("s") * _NC + lax.axis_index("c")
    base = wid * _PER_W
    ebase = row_base + base
    pltpu.sync_copy(labs_hbm.at[0], idx_v)
    w_copy = pltpu.async_copy(
        w_hbm.at[pl.ds(base, _PER_W)], w_v, sem_w)

    cbuf = (c_v0, c_v1)
    ebuf = (e_v0, e_v1)
    csem = (sem_c0, sem_c1)
    esem = (sem_e0, sem_e1)

    def start(g):
        slot = g % _NBUF
        dc = pltpu.async_copy(
            centers_hbm.at[idx_v.at[pl.ds(base + g * _CHUNK, _CHUNK)]],
            cbuf[slot], csem[slot])
        de = pltpu.async_copy(
            emb_hbm.at[pl.ds(ebase + g * _CHUNK, _CHUNK)],
            ebuf[slot], esem[slot])
        return dc, de

    pend = {g: start(g) for g in range(_NBUF - 1)}
    w_copy.wait()
    acc = jnp.zeros((16,), jnp.float32)
    wacc = jnp.zeros((16,), jnp.float32)
    for g in range(_NCHUNK):
        if g + _NBUF - 1 < _NCHUNK:
            pend[g + _NBUF - 1] = start(g + _NBUF - 1)
        pend[g][0].wait()
        pend[g][1].wait()
        c_v = cbuf[g % _NBUF]
        e_v = ebuf[g % _NBUF]

        def row_body(r, carry):
            acc, wacc = carry
            wspl = w_v[g * _CHUNK + r, :]
            s = jnp.zeros((16,), jnp.float32)
            for k in range(_D // 16):
                ev = e_v[r, pl.ds(k * 16, 16)]
                cv = c_v[r, pl.ds(k * 16, 16)]
                d = ev - cv
                s = s + d * d
            return acc + wspl * s, wacc + wspl

        acc, wacc = lax.fori_loop(0, _CHUNK, row_body, (acc, wacc))

    res_v[0, :] = acc
    res_v[1, :] = wacc
    pltpu.sync_copy(res_v.at[0], out_sq_hbm.at[wid])
    pltpu.sync_copy(res_v.at[1], out_w_hbm.at[wid])


def _sc_loss(centers, labs_row, w16, emb_flat, row_base):
    mesh = plsc.VectorSubcoreMesh(
        core_axis_name="c", subcore_axis_name="s")
    run = pl.kernel(
        functools.partial(_sc_body, row_base),
        out_type=[
            jax.ShapeDtypeStruct((_NW, 16), jnp.float32),
            jax.ShapeDtypeStruct((_NW, 16), jnp.float32),
        ],
        mesh=mesh,
        scratch_types=[
            pltpu.VMEM((_NROWS,), jnp.int32),
            pltpu.VMEM((_PER_W, 16), jnp.float32),
            pltpu.VMEM((_CHUNK, _D), jnp.float32),
            pltpu.VMEM((_CHUNK, _D), jnp.float32),
            pltpu.VMEM((_CHUNK, _D), jnp.float32),
            pltpu.VMEM((_CHUNK, _D), jnp.float32),
            pltpu.VMEM((2, 16), jnp.float32),
            pltpu.SemaphoreType.DMA,
            pltpu.SemaphoreType.DMA,
            pltpu.SemaphoreType.DMA,
            pltpu.SemaphoreType.DMA,
            pltpu.SemaphoreType.DMA,
        ],
    )
    return run(centers, labs_row, w16, emb_flat)


@jax.jit
def kernel(predicts, embedding, labels, label_len, centers):
    predicts2 = predicts.reshape(_N, _C)
    emb_flat = embedding.reshape(_N, _D)

    parts = []
    for h in range(_NH):
        labs, w16 = _prep(predicts2, labels, label_len, h)
        part_sq, part_w = _sc_loss(
            centers, labs, w16, emb_flat, h * _NROWS)
        parts.append((part_sq, part_w))

    cat = jnp.concatenate([p[0] for p in parts] + [p[1] for p in parts])
    s = jnp.sum(cat.reshape(2, _NH * _NW * 16 // 1), axis=1)
    total = s[0]
    wsum = s[1] / 16.0
    return total / (wsum * _D)


# speedup vs baseline: 1.1254x; 1.0044x over previous
import functools

import jax
import jax.numpy as jnp
from jax import lax
from jax.experimental import pallas as pl
from jax.experimental.pallas import tpu as pltpu
from jax.experimental.pallas import tpu_sc as plsc

_C = 6625
_D = 512
_B = 64
_S = 80
_N = _B * _S
_NH = 1
_NROWS = _N // _NH
_NEG = -3.4e38


_R1 = 640
_G1 = _NROWS // _R1


def _prep_body(ll_ref, p_ref, labels_ref, labs_ref, w_ref):
    step = pl.program_id(0)
    x = p_ref[...]
    m = jnp.max(x, axis=1, keepdims=True)
    ci = lax.broadcasted_iota(jnp.int32, (_R1, _C), 1)
    raw = jnp.min(jnp.where(x == m, ci, _C), axis=1, keepdims=True)

    ri = lax.broadcasted_iota(jnp.int32, (_R1, 1), 0)
    prev = jnp.concatenate(
        [jnp.full((1, 1), -1, jnp.int32), raw[:-1]], axis=0)
    first = (ri % _S) == 0
    char_rep = jnp.logical_and(prev == raw, jnp.logical_not(first))
    mk = jnp.logical_and(raw > 0, jnp.logical_not(char_rep)
                         ).astype(jnp.float32)

    rr = lax.broadcasted_iota(jnp.int32, (_R1, _R1), 0)
    tt = lax.broadcasted_iota(jnp.int32, (_R1, _R1), 1)
    sameseg = (rr // _S) == (tt // _S)
    seg_f = sameseg.astype(jnp.float32)
    ltri = jnp.where(tt <= rr, seg_f, 0.0)

    cs = jnp.dot(ltri, mk, preferred_element_type=jnp.float32)
    cnt = jnp.dot(seg_f, mk, preferred_element_type=jnp.float32)

    nsamp = _R1 // _S
    ra = lax.broadcasted_iota(jnp.int32, (_R1, nsamp), 0)
    ca = lax.broadcasted_iota(jnp.int32, (_R1, nsamp), 1)
    amat = ((ra // _S) == ca).astype(jnp.float32)
    ll_col = jnp.concatenate(
        [jnp.full((_S, 1), 1.0, jnp.float32)
         * ll_ref[step * nsamp + i].astype(jnp.float32)
         for i in range(nsamp)], axis=0)
    lab_rows = jnp.dot(amat, labels_ref[...].astype(jnp.float32),
                       preferred_element_type=jnp.float32)
    t80 = lax.broadcasted_iota(jnp.int32, (_R1, _S), 1)
    r80 = lax.broadcasted_iota(jnp.int32, (_R1, _S), 0) % _S
    labels_col = jnp.sum(jnp.where(t80 == r80, lab_rows, 0.0),
                         axis=1, keepdims=True)

    valid = (cnt == ll_col).astype(jnp.float32)
    rank = jnp.clip(cs.astype(jnp.int32) - 1, 0, _S - 1)

    tloc = tt % _S
    onehot = jnp.where(tloc == rank, seg_f, 0.0)
    labs_f = jnp.dot(onehot, labels_col,
                     preferred_element_type=jnp.float32)

    ident = (rr == tt).astype(jnp.float32)
    labs_row = lax.dot_general(
        labs_f, ident, (((0,), (0,)), ((), ())),
        preferred_element_type=jnp.float32)
    labs_ref[...] = labs_row.astype(jnp.int32)
    w_ref[...] = jnp.broadcast_to(mk * valid, (_R1, 16))


def _prep(predicts2, labels2, ll_rep, half):
    base = half * _G1
    return pl.pallas_call(
        _prep_body,
        grid=(_G1,),
        in_specs=[
            pl.BlockSpec(memory_space=pltpu.SMEM),
            pl.BlockSpec((_R1, _C), lambda r: (base + r, 0)),
            pl.BlockSpec((_R1 // _S, _S), lambda r: (base + r, 0)),
        ],
        out_specs=[
            pl.BlockSpec((1, _R1), lambda r: (0, r)),
            pl.BlockSpec((_R1, 16), lambda r: (r, 0)),
        ],
        out_shape=[
            jax.ShapeDtypeStruct((1, _NROWS), jnp.int32),
            jax.ShapeDtypeStruct((_NROWS, 16), jnp.float32),
        ],
    )(ll_rep, predicts2, labels2)


_NC, _NS = 2, 16
_NW = _NC * _NS
_PER_W = _NROWS // _NW
_CHUNK = 40
_NCHUNK = _PER_W // _CHUNK
_NBUF = 2


def _sc_body(row_base, centers_hbm, labs_hbm, w_hbm, emb_hbm,
             out_sq_hbm, out_w_hbm,
             idx_v, w_v, c_v0, c_v1, e_v0, e_v1, res_v,
             sem_c0, sem_c1, sem_e0, sem_e1, sem_w):
    wid = lax.axis_index("s") * _NC + lax.axis_index("c")
    base = wid * _PER_W
    ebase = row_base + base
    pltpu.sync_copy(labs_hbm.at[0], idx_v)
    w_copy = pltpu.async_copy(
        w_hbm.at[pl.ds(base, _PER_W)], w_v, sem_w)

    cbuf = (c_v0, c_v1)
    ebuf = (e_v0, e_v1)
    csem = (sem_c0, sem_c1)
    esem = (sem_e0, sem_e1)

    def start(g):
        slot = g % _NBUF
        dc = pltpu.async_copy(
            centers_hbm.at[idx_v.at[pl.ds(base + g * _CHUNK, _CHUNK)]],
            cbuf[slot], csem[slot])
        de = pltpu.async_copy(
            emb_hbm.at[pl.ds(ebase + g * _CHUNK, _CHUNK)],
            ebuf[slot], esem[slot])
        return dc, de

    pend = {g: start(g) for g in range(_NBUF - 1)}
    w_copy.wait()
    acc = jnp.zeros((16,), jnp.float32)
    wacc = jnp.zeros((16,), jnp.float32)
    for g in range(_NCHUNK):
        if g + _NBUF - 1 < _NCHUNK:
            pend[g + _NBUF - 1] = start(g + _NBUF - 1)
        pend[g][0].wait()
        pend[g][1].wait()
        c_v = cbuf[g % _NBUF]
        e_v = ebuf[g % _NBUF]

        def row_body(r, carry):
            acc, wacc = carry
            wspl = w_v[g * _CHUNK + r, :]
            s = jnp.zeros((16,), jnp.float32)
            for k in range(_D // 16):
                ev = e_v[r, pl.ds(k * 16, 16)]
                cv = c_v[r, pl.ds(k * 16, 16)]
                d = ev - cv
                s = s + d * d
            return acc + wspl * s, wacc + wspl

        acc, wacc = plsc.parallel_loop(
            0, _CHUNK, 1, unroll=4, carry=(acc, wacc))(row_body)

    res_v[0, :] = acc
    res_v[1, :] = wacc
    pltpu.sync_copy(res_v.at[0], out_sq_hbm.at[wid])
    pltpu.sync_copy(res_v.at[1], out_w_hbm.at[wid])


def _sc_loss(centers, labs_row, w16, emb_flat, row_base):
    mesh = plsc.VectorSubcoreMesh(
        core_axis_name="c", subcore_axis_name="s")
    run = pl.kernel(
        functools.partial(_sc_body, row_base),
        out_type=[
            jax.ShapeDtypeStruct((_NW, 16), jnp.float32),
            jax.ShapeDtypeStruct((_NW, 16), jnp.float32),
        ],
        mesh=mesh,
        scratch_types=[
            pltpu.VMEM((_NROWS,), jnp.int32),
            pltpu.VMEM((_PER_W, 16), jnp.float32),
            pltpu.VMEM((_CHUNK, _D), jnp.float32),
            pltpu.VMEM((_CHUNK, _D), jnp.float32),
            pltpu.VMEM((_CHUNK, _D), jnp.float32),
            pltpu.VMEM((_CHUNK, _D), jnp.float32),
            pltpu.VMEM((2, 16), jnp.float32),
            pltpu.SemaphoreType.DMA,
            pltpu.SemaphoreType.DMA,
            pltpu.SemaphoreType.DMA,
            pltpu.SemaphoreType.DMA,
            pltpu.SemaphoreType.DMA,
        ],
    )
    return run(centers, labs_row, w16, emb_flat)


@jax.jit
def kernel(predicts, embedding, labels, label_len, centers):
    predicts2 = predicts.reshape(_N, _C)
    emb_flat = embedding.reshape(_N, _D)

    parts = []
    for h in range(_NH):
        labs, w16 = _prep(predicts2, labels, label_len, h)
        part_sq, part_w = _sc_loss(
            centers, labs, w16, emb_flat, h * _NROWS)
        parts.append((part_sq, part_w))

    cat = jnp.concatenate([p[0] for p in parts] + [p[1] for p in parts])
    s = jnp.sum(cat.reshape(2, _NH * _NW * 16 // 1), axis=1)
    total = s[0]
    wsum = s[1] / 16.0
    return total / (wsum * _D)
